# baseline (speedup 1.0000x reference)
import functools

import jax
import jax.numpy as jnp
from jax import lax
from jax.experimental import pallas as pl
from jax.experimental.pallas import tpu as pltpu
from jax.experimental.pallas import tpu_sc as plsc

VOCAB = 100000
D = 64
WPAD = 128
BATCH = 4096
SEQ = 50
LANES = 16

NC = 2
NS = 16
NW = NC * NS

TOTAL = BATCH * SEQ
PER_W = TOTAL // NW
CHUNK = 128
NCHUNK = PER_W // CHUNK
NBUF = 2
OUT_COLS = 128
OUT_ROWS = TOTAL * D // OUT_COLS
WB_ROWS = CHUNK * D // OUT_COLS
POS_EXT = SEQ + CHUNK - 1


def _pos_ext_table():
    pos = jnp.arange(SEQ, dtype=jnp.float32)[:, None]
    i = jnp.arange(D, dtype=jnp.float32)[None, :]
    angle = pos / jnp.power(10000.0, 2.0 * jnp.floor(i / 2.0) / D)
    angle = angle.at[:, 0::2].set(jnp.sin(angle[:, 0::2]))
    angle = angle.at[:, 1::2].set(jnp.cos(angle[:, 1::2]))
    return jnp.tile(angle, (4, 1))[:POS_EXT]


def _sc_embed(wpad, idx2d, pos_ext):
    mesh = plsc.VectorSubcoreMesh(core_axis_name="c", subcore_axis_name="s")

    @functools.partial(
        pl.kernel,
        mesh=mesh,
        compiler_params=pltpu.CompilerParams(use_tc_tiling_on_sc=False),
        out_type=jax.ShapeDtypeStruct((OUT_ROWS, OUT_COLS), jnp.float32),
        scratch_types=[
            pltpu.VMEM((NCHUNK, CHUNK), jnp.int32),
            pltpu.VMEM((POS_EXT, D), jnp.float32),
            pltpu.VMEM((NBUF, CHUNK, WPAD), jnp.float32),
            pltpu.VMEM((NBUF, WB_ROWS, OUT_COLS), jnp.float32),
            pltpu.SemaphoreType.DMA((NBUF,)),
            pltpu.SemaphoreType.DMA((NBUF,)),
        ],
    )
    def k(w_hbm, idx_hbm, pos_hbm, out_hbm, idx_v, pos_v, gbuf, wbuf, gsem,
          wsem):
        wid = lax.axis_index("s") * NC + lax.axis_index("c")
        pltpu.sync_copy(idx_hbm.at[pl.ds(wid * NCHUNK, NCHUNK)], idx_v)
        pltpu.sync_copy(pos_hbm, pos_v)
        base_flat = wid * PER_W
        out_base = wid * (PER_W * D // OUT_COLS)

        for b in range(NBUF):
            pltpu.async_copy(w_hbm.at[idx_v.at[b]], gbuf.at[b], gsem.at[b])

        def pair_body(i, carry):
            for b in range(NBUF):
                c = i * NBUF + b
                pltpu.make_async_copy(w_hbm.at[pl.ds(0, CHUNK)], gbuf.at[b],
                                      gsem.at[b]).wait()
                @pl.when(i > 0)
                def _():
                    pltpu.make_async_copy(wbuf.at[b],
                                          out_hbm.at[pl.ds(0, WB_ROWS)],
                                          wsem.at[b]).wait()

                phase = lax.rem(base_flat + c * CHUNK, SEQ)
                for r in range(CHUNK):
                    wr = r // 2
                    wc = (r % 2) * D
                    for j in range(D // LANES):
                        sl = pl.ds(j * LANES, LANES)
                        wbuf[b, wr, pl.ds(wc + j * LANES, LANES)] = (
                            gbuf[b, r, sl] * 0.125 + pos_v[phase + r, sl])

                pltpu.async_copy(wbuf.at[b],
                                 out_hbm.at[pl.ds(out_base + c * WB_ROWS,
                                                  WB_ROWS)],
                                 wsem.at[b])

                @pl.when(i < NCHUNK // NBUF - 1)
                def _():
                    pltpu.async_copy(w_hbm.at[idx_v.at[c + NBUF]],
                                     gbuf.at[b], gsem.at[b])
            return carry

        lax.fori_loop(0, NCHUNK // NBUF, pair_body, 0)

        for b in range(NBUF):
            pltpu.make_async_copy(wbuf.at[b], out_hbm.at[pl.ds(0, WB_ROWS)],
                                  wsem.at[b]).wait()

    return k(wpad, idx2d, pos_ext)


def kernel(input, weight):
    wpad = jnp.pad(weight, ((0, 0), (0, WPAD - D)))
    idx2d = input.reshape(TOTAL // CHUNK, CHUNK)
    pos_ext = _pos_ext_table()
    out = _sc_embed(wpad, idx2d, pos_ext)
    return out.reshape(BATCH, SEQ, D)


# speedup vs baseline: 1.5310x; 1.5310x over previous
import functools

import jax
import jax.numpy as jnp
from jax import lax
from jax.experimental import pallas as pl
from jax.experimental.pallas import tpu as pltpu
from jax.experimental.pallas import tpu_sc as plsc

VOCAB = 100000
D = 64
BATCH = 4096
SEQ = 50
LANES = 16

NC = 2
NS = 16
NW = NC * NS

BATCH_W = BATCH // NW
DMA_ROWS = 100
SEQ_CHUNK = 8
CHUNK = SEQ_CHUNK * SEQ
Q = CHUNK // DMA_ROWS
NCHUNK = BATCH_W // SEQ_CHUNK
IDX_ROWS_W = BATCH_W * SEQ // DMA_ROWS
NBUF = 2


def _pos_table():
    pos = jnp.arange(SEQ, dtype=jnp.float32)[:, None]
    i = jnp.arange(D, dtype=jnp.float32)[None, :]
    angle = pos / jnp.power(10000.0, 2.0 * jnp.floor(i / 2.0) / D)
    angle = angle.at[:, 0::2].set(jnp.sin(angle[:, 0::2]))
    angle = angle.at[:, 1::2].set(jnp.cos(angle[:, 1::2]))
    return angle


def _sc_embed(weight, idx2d, pos):
    mesh = plsc.VectorSubcoreMesh(core_axis_name="c", subcore_axis_name="s")

    @functools.partial(
        pl.kernel,
        mesh=mesh,
        compiler_params=pltpu.CompilerParams(use_tc_tiling_on_sc=False),
        out_type=jax.ShapeDtypeStruct((BATCH, SEQ, D), jnp.float32),
        scratch_types=[
            pltpu.VMEM((IDX_ROWS_W, DMA_ROWS), jnp.int32),
            pltpu.VMEM((SEQ, D), jnp.float32),
            pltpu.VMEM((NBUF, CHUNK, D), jnp.float32),
            pltpu.VMEM((NBUF, SEQ_CHUNK, SEQ, D), jnp.float32),
            pltpu.SemaphoreType.DMA((NBUF,)),
            pltpu.SemaphoreType.DMA((NBUF,)),
        ],
    )
    def k(w_hbm, idx_hbm, pos_hbm, out_hbm, idx_v, pos_v, gbuf, wbuf, gsem,
          wsem):
        wid = lax.axis_index("s") * NC + lax.axis_index("c")
        pltpu.sync_copy(idx_hbm.at[pl.ds(wid * IDX_ROWS_W, IDX_ROWS_W)],
                        idx_v)
        pltpu.sync_copy(pos_hbm, pos_v)
        out_w = wid * BATCH_W

        def fire_gathers(c, b):
            for q in range(Q):
                pltpu.async_copy(
                    w_hbm.at[idx_v.at[c * Q + q]],
                    gbuf.at[b].at[pl.ds(q * DMA_ROWS, DMA_ROWS)],
                    gsem.at[b])

        for b in range(NBUF):
            fire_gathers(b, b)

        def pair_body(i, carry):
            for b in range(NBUF):
                c = i * NBUF + b
                pltpu.make_async_copy(w_hbm.at[pl.ds(0, CHUNK)], gbuf.at[b],
                                      gsem.at[b]).wait()
                @pl.when(i > 0)
                def _():
                    pltpu.make_async_copy(wbuf.at[b],
                                          out_hbm.at[pl.ds(0, SEQ_CHUNK)],
                                          wsem.at[b]).wait()

                @plsc.parallel_loop(0, SEQ, unroll=5)
                def _(s):
                    pv = [pos_v[s, pl.ds(j * LANES, LANES)]
                          for j in range(D // LANES)]
                    for t in range(SEQ_CHUNK):
                        for j in range(D // LANES):
                            sl = pl.ds(j * LANES, LANES)
                            wbuf[b, t, s, sl] = (
                                gbuf[b, t * SEQ + s, sl] * 0.125 + pv[j])

                pltpu.async_copy(
                    wbuf.at[b],
                    out_hbm.at[pl.ds(out_w + c * SEQ_CHUNK, SEQ_CHUNK)],
                    wsem.at[b])

                @pl.when(i < NCHUNK // NBUF - 1)
                def _():
                    fire_gathers(c + NBUF, b)
            return carry

        lax.fori_loop(0, NCHUNK // NBUF, pair_body, 0)

        for b in range(NBUF):
            pltpu.make_async_copy(wbuf.at[b], out_hbm.at[pl.ds(0, SEQ_CHUNK)],
                                  wsem.at[b]).wait()

    return k(weight, idx2d, pos)


def kernel(input, weight):
    idx2d = input.reshape(BATCH * SEQ // DMA_ROWS, DMA_ROWS)
    pos = _pos_table()
    return _sc_embed(weight, idx2d, pos)


# speedup vs baseline: 1.9123x; 1.2491x over previous
import functools

import jax
import jax.numpy as jnp
from jax import lax
from jax.experimental import pallas as pl
from jax.experimental.pallas import tpu as pltpu
from jax.experimental.pallas import tpu_sc as plsc

VOCAB = 100000
D = 64
WPAD = 128
BATCH = 4096
SEQ = 50
LANES = 16

NC = 2
NS = 16
NW = NC * NS

BATCH_W = BATCH // NW
SEQ_CHUNK = 4
CHUNK = SEQ_CHUNK * SEQ
DMA_ROWS = 40
Q = CHUNK // DMA_ROWS
NCHUNK = BATCH_W // SEQ_CHUNK
IDX_ROWS_W = BATCH_W * SEQ // DMA_ROWS
NBUF = 2


def _pos_table():
    pos = jnp.arange(SEQ, dtype=jnp.float32)[:, None]
    i = jnp.arange(D, dtype=jnp.float32)[None, :]
    angle = pos / jnp.power(10000.0, 2.0 * jnp.floor(i / 2.0) / D)
    angle = angle.at[:, 0::2].set(jnp.sin(angle[:, 0::2]))
    angle = angle.at[:, 1::2].set(jnp.cos(angle[:, 1::2]))
    return angle


def _sc_embed(wpad, idx2d, pos):
    mesh = plsc.VectorSubcoreMesh(core_axis_name="c", subcore_axis_name="s")

    @functools.partial(
        pl.kernel,
        mesh=mesh,
        compiler_params=pltpu.CompilerParams(use_tc_tiling_on_sc=True),
        out_type=jax.ShapeDtypeStruct((BATCH, SEQ, D), jnp.float32),
        scratch_types=[
            pltpu.VMEM((NBUF, Q, DMA_ROWS), jnp.int32),
            pltpu.VMEM((SEQ, D), jnp.float32),
            pltpu.VMEM((NBUF, CHUNK, WPAD), jnp.float32),
            pltpu.VMEM((NBUF, SEQ_CHUNK, SEQ, D), jnp.float32),
            pltpu.SemaphoreType.DMA((NBUF,)),
            pltpu.SemaphoreType.DMA((NBUF,)),
            pltpu.SemaphoreType.DMA((NBUF,)),
        ],
    )
    def k(w_hbm, idx_hbm, pos_hbm, out_hbm, idx_v, pos_v, gbuf, wbuf, isem,
          gsem, wsem):
        wid = lax.axis_index("s") * NC + lax.axis_index("c")
        pltpu.sync_copy(pos_hbm, pos_v)
        out_w = wid * BATCH_W
        idx_w = wid * (IDX_ROWS_W // Q)

        def stage_idx(c, b):
            pltpu.async_copy(idx_hbm.at[idx_w + c], idx_v.at[b], isem.at[b])

        def fire_gathers(b):
            pltpu.make_async_copy(idx_hbm.at[pl.ds(0, 1)],
                                  idx_v.at[b], isem.at[b]).wait()
            for q in range(Q):
                pltpu.async_copy(
                    w_hbm.at[idx_v.at[b].at[q]],
                    gbuf.at[b].at[pl.ds(q * DMA_ROWS, DMA_ROWS)],
                    gsem.at[b])

        for b in range(NBUF):
            stage_idx(b, b)
        for b in range(NBUF):
            fire_gathers(b)

        def pair_body(i, carry):
            for b in range(NBUF):
                c = i * NBUF + b
                pltpu.make_async_copy(w_hbm.at[pl.ds(0, CHUNK)], gbuf.at[b],
                                      gsem.at[b]).wait()

                @pl.when(i < NCHUNK // NBUF - 1)
                def _():
                    stage_idx(c + NBUF, b)

                @pl.when(i > 0)
                def _():
                    pltpu.make_async_copy(wbuf.at[b],
                                          out_hbm.at[pl.ds(0, SEQ_CHUNK)],
                                          wsem.at[b]).wait()

                @plsc.parallel_loop(0, SEQ, unroll=5)
                def _(s):
                    pv = [pos_v[s, pl.ds(j * LANES, LANES)]
                          for j in range(D // LANES)]
                    for t in range(SEQ_CHUNK):
                        for j in range(D // LANES):
                            sl = pl.ds(j * LANES, LANES)
                            wbuf[b, t, s, sl] = (
                                gbuf[b, t * SEQ + s, sl] * 0.125 + pv[j])

                pltpu.async_copy(
                    wbuf.at[b],
                    out_hbm.at[pl.ds(out_w + c * SEQ_CHUNK, SEQ_CHUNK)],
                    wsem.at[b])

                @pl.when(i < NCHUNK // NBUF - 1)
                def _():
                    fire_gathers(b)
            return carry

        lax.fori_loop(0, NCHUNK // NBUF, pair_body, 0)

        for b in range(NBUF):
            pltpu.make_async_copy(wbuf.at[b], out_hbm.at[pl.ds(0, SEQ_CHUNK)],
                                  wsem.at[b]).wait()

    return k(wpad, idx2d, pos)


def kernel(input, weight):
    wpad = jnp.pad(weight, ((0, 0), (0, WPAD - D)))
    idx3d = input.reshape(BATCH * SEQ // CHUNK, Q, DMA_ROWS)
    pos = _pos_table()
    return _sc_embed(wpad, idx3d, pos)
